# Initial kernel scaffold; baseline (speedup 1.0000x reference)
import jax
import jax.numpy as jnp
from jax.experimental import pallas as pl


def kernel(hidden_states, W_sel):
    raise NotImplementedError("write your pallas kernel here")


# speedup vs baseline: 7.6227x; 7.6227x over previous
import functools

import jax
import jax.numpy as jnp
from jax.experimental import pallas as pl
from jax.experimental.pallas import tpu as pltpu

D_MODEL = 768
E = 64
CHUNK = 2048


_SIGN = -2147483648


def _key_of(a):
    ki = jax.lax.bitcast_convert_type(a, jnp.int32)
    ku = jax.lax.bitcast_convert_type(a, jnp.uint32)
    kun = jnp.where(ki < 0, ~ku, ku | jnp.uint32(0x80000000))
    return jax.lax.bitcast_convert_type(kun, jnp.int32) ^ jnp.int32(_SIGN)


def _unkey(k):
    ku = jax.lax.bitcast_convert_type(k ^ jnp.int32(_SIGN), jnp.uint32)
    top = (ku & jnp.uint32(0x80000000)) != 0
    u = jnp.where(top, ku ^ jnp.uint32(0x80000000), ~ku)
    return jax.lax.bitcast_convert_type(u, jnp.float32)


def _matmul_body(x_ref, w_ref, key_ref):
    aff = jax.lax.dot_general(
        x_ref[...], w_ref[...],
        dimension_numbers=(((1,), (1,)), ((), ())),
        preferred_element_type=jnp.float32,
    )
    key_ref[...] = _key_of(aff)


def _affinity_key(tokens, w_sel, *, interpret=False):
    n, d = tokens.shape
    T = min(4096, n)
    return pl.pallas_call(
        _matmul_body,
        grid=(n // T,),
        in_specs=[
            pl.BlockSpec((T, d), lambda i: (i, 0)),
            pl.BlockSpec((E, d), lambda i: (0, 0)),
        ],
        out_specs=pl.BlockSpec((T, E), lambda i: (i, 0)),
        out_shape=jax.ShapeDtypeStruct((n, E), jnp.int32),
        interpret=interpret,
    )(tokens, w_sel)


def _halves_sum(x):
    s = x[:, :E] + x[:, E:]
    return jnp.concatenate([s, s], axis=1)


def _halves_max(x):
    s = jnp.maximum(x[:, :E], x[:, E:])
    return jnp.concatenate([s, s], axis=1)


def _router_body(key_ref, w_out_ref, s_out_ref, *, cap):
    n2 = key_ref.shape[0]
    chunk = min(CHUNK, n2)
    nchunks = n2 // chunk
    L = 2 * E

    def max_step(c, m):
        k = key_ref[pl.ds(c * chunk, chunk), :]
        return jnp.maximum(m, jnp.max(k, axis=0, keepdims=True))

    kmax = jax.lax.fori_loop(0, nchunks, max_step,
                             jnp.full((1, L), _SIGN, jnp.int32))
    mx = _unkey(_halves_max(kmax))

    def bit_step(b, lo):
        bit = jnp.int32(1) << (31 - b)
        cand = lo | bit
        cand_s = cand ^ jnp.int32(_SIGN)

        def cnt_step(c, acc):
            k = key_ref[pl.ds(c * chunk, chunk), :]
            return acc + jnp.sum((k >= cand_s).astype(jnp.int32),
                                 axis=0, keepdims=True)

        cnt = jax.lax.fori_loop(0, nchunks, cnt_step,
                                jnp.zeros((1, L), jnp.int32))
        cnt = _halves_sum(cnt)
        return jnp.where(cnt >= cap, cand, lo)

    lo_pat = jax.lax.fori_loop(0, 32, bit_step, jnp.zeros((1, L), jnp.int32))
    thr = lo_pat ^ jnp.int32(_SIGN)

    def sum_step(c, acc):
        k = key_ref[pl.ds(c * chunk, chunk), :]
        sel = k >= thr
        p = jnp.where(sel, jnp.exp(_unkey(k) - mx), 0.0)
        return acc + jnp.sum(p, axis=0, keepdims=True)

    s = _halves_sum(jax.lax.fori_loop(0, nchunks, sum_step,
                                      jnp.zeros((1, L), jnp.float32)))

    lane = jax.lax.broadcasted_iota(jnp.int32, (1, L), 1)
    half0 = lane < E

    def out_step(c, carry):
        rows = pl.ds(c * chunk, chunk)
        k = key_ref[rows, :]
        sel = k >= thr
        self_f = sel.astype(jnp.float32)
        p = jnp.where(sel, jnp.exp(_unkey(k) - mx), 0.0)
        cnt0 = jnp.sum(jnp.where(half0, self_f, 0.0), axis=1, keepdims=True)
        cnt1 = jnp.sum(jnp.where(half0, 0.0, self_f), axis=1, keepdims=True)
        div = jnp.maximum(jnp.where(half0, cnt0, cnt1), 1.0)
        w_out_ref[rows, :] = p / (s * div)
        s_out_ref[rows, :] = self_f
        return carry

    jax.lax.fori_loop(0, nchunks, out_step, 0)


def _route(key_folded, cap, *, interpret=False):
    n2, l = key_folded.shape
    return pl.pallas_call(
        functools.partial(_router_body, cap=cap),
        out_shape=(
            jax.ShapeDtypeStruct((n2, l), jnp.float32),
            jax.ShapeDtypeStruct((n2, l), jnp.float32),
        ),
        interpret=interpret,
    )(key_folded)


def kernel(hidden_states, W_sel):
    batch, seq, d = hidden_states.shape
    num_tokens = batch * seq
    cap = int(num_tokens / E)
    c = min(cap, num_tokens)
    tokens = hidden_states.reshape(num_tokens, d)
    key = _affinity_key(tokens, W_sel)
    key_folded = key.reshape(num_tokens // 2, 2 * E)
    w_f, a_f = _route(key_folded, c)
    weights = w_f.reshape(num_tokens, E)
    assign = a_f.reshape(num_tokens, E)
    return (weights, assign, cap)


# speedup vs baseline: 9.3900x; 1.2318x over previous
import functools

import jax
import jax.numpy as jnp
from jax.experimental import pallas as pl
from jax.experimental.pallas import tpu as pltpu

D_MODEL = 768
E = 64
CHUNK = 2048


_SIGN = -2147483648


def _key_of(a):
    ki = jax.lax.bitcast_convert_type(a, jnp.int32)
    ku = jax.lax.bitcast_convert_type(a, jnp.uint32)
    kun = jnp.where(ki < 0, ~ku, ku | jnp.uint32(0x80000000))
    return jax.lax.bitcast_convert_type(kun, jnp.int32) ^ jnp.int32(_SIGN)


def _unkey(k):
    ku = jax.lax.bitcast_convert_type(k ^ jnp.int32(_SIGN), jnp.uint32)
    top = (ku & jnp.uint32(0x80000000)) != 0
    u = jnp.where(top, ku ^ jnp.uint32(0x80000000), ~ku)
    return jax.lax.bitcast_convert_type(u, jnp.float32)


def _matmul_body(x0_ref, x1_ref, w_ref, key_ref):
    dn = (((1,), (1,)), ((), ()))
    a0 = jax.lax.dot_general(x0_ref[...], w_ref[...], dimension_numbers=dn,
                             preferred_element_type=jnp.float32)
    a1 = jax.lax.dot_general(x1_ref[...], w_ref[...], dimension_numbers=dn,
                             preferred_element_type=jnp.float32)
    key_ref[...] = jnp.concatenate([_key_of(a0), _key_of(a1)], axis=1)


def _affinity_key(tokens, w_sel, *, interpret=False):
    n, d = tokens.shape
    half = n // 2
    T = min(2048, half)
    nb = half // T
    return pl.pallas_call(
        _matmul_body,
        grid=(nb,),
        in_specs=[
            pl.BlockSpec((T, d), lambda i: (i, 0)),
            pl.BlockSpec((T, d), lambda i: (i + nb, 0)),
            pl.BlockSpec((E, d), lambda i: (0, 0)),
        ],
        out_specs=pl.BlockSpec((T, 2 * E), lambda i: (i, 0)),
        out_shape=jax.ShapeDtypeStruct((half, 2 * E), jnp.int32),
        interpret=interpret,
    )(tokens, tokens, w_sel)


def _halves_sum(x):
    s = x[:, :E] + x[:, E:]
    return jnp.concatenate([s, s], axis=1)


def _halves_max(x):
    s = jnp.maximum(x[:, :E], x[:, E:])
    return jnp.concatenate([s, s], axis=1)


def _router_body(key_ref, w_out_ref, s_out_ref, *, cap):
    n2 = key_ref.shape[0]
    chunk = min(CHUNK, n2)
    nchunks = n2 // chunk
    L = 2 * E

    def max_step(c, m):
        k = key_ref[pl.ds(c * chunk, chunk), :]
        return jnp.maximum(m, jnp.max(k, axis=0, keepdims=True))

    kmax = jax.lax.fori_loop(0, nchunks, max_step,
                             jnp.full((1, L), _SIGN, jnp.int32))
    mx = _unkey(_halves_max(kmax))

    def bit_step(b, lo):
        bit = jnp.int32(1) << (31 - b)
        cand = lo | bit
        cand_s = cand ^ jnp.int32(_SIGN)

        def cnt_step(c, acc):
            k = key_ref[pl.ds(c * chunk, chunk), :]
            return acc + jnp.sum((k >= cand_s).astype(jnp.int32),
                                 axis=0, keepdims=True)

        cnt = jax.lax.fori_loop(0, nchunks, cnt_step,
                                jnp.zeros((1, L), jnp.int32))
        cnt = _halves_sum(cnt)
        return jnp.where(cnt >= cap, cand, lo)

    lo_pat = jax.lax.fori_loop(0, 32, bit_step, jnp.zeros((1, L), jnp.int32))
    thr = lo_pat ^ jnp.int32(_SIGN)

    def sum_step(c, acc):
        k = key_ref[pl.ds(c * chunk, chunk), :]
        sel = k >= thr
        p = jnp.where(sel, jnp.exp(_unkey(k) - mx), 0.0)
        return acc + jnp.sum(p, axis=0, keepdims=True)

    s = _halves_sum(jax.lax.fori_loop(0, nchunks, sum_step,
                                      jnp.zeros((1, L), jnp.float32)))

    lane = jax.lax.broadcasted_iota(jnp.int32, (1, L), 1)
    half0 = lane < E

    def out_step(c, carry):
        rows = pl.ds(c * chunk, chunk)
        k = key_ref[rows, :]
        sel = k >= thr
        self_f = sel.astype(jnp.float32)
        p = jnp.where(sel, jnp.exp(_unkey(k) - mx), 0.0)
        cnt0 = jnp.sum(jnp.where(half0, self_f, 0.0), axis=1, keepdims=True)
        cnt1 = jnp.sum(jnp.where(half0, 0.0, self_f), axis=1, keepdims=True)
        div = jnp.maximum(jnp.where(half0, cnt0, cnt1), 1.0)
        w = p / (s * div)
        r0 = pl.ds(c * chunk, chunk)
        r1 = pl.ds(n2 + c * chunk, chunk)
        w_out_ref[r0, :] = w[:, :E]
        w_out_ref[r1, :] = w[:, E:]
        s_out_ref[r0, :] = self_f[:, :E]
        s_out_ref[r1, :] = self_f[:, E:]
        return carry

    jax.lax.fori_loop(0, nchunks, out_step, 0)


def _route(key_folded, cap, *, interpret=False):
    n2, l = key_folded.shape
    n = n2 * 2
    return pl.pallas_call(
        functools.partial(_router_body, cap=cap),
        out_shape=(
            jax.ShapeDtypeStruct((n, E), jnp.float32),
            jax.ShapeDtypeStruct((n, E), jnp.float32),
        ),
        interpret=interpret,
    )(key_folded)


def kernel(hidden_states, W_sel):
    batch, seq, d = hidden_states.shape
    num_tokens = batch * seq
    cap = int(num_tokens / E)
    c = min(cap, num_tokens)
    tokens = hidden_states.reshape(num_tokens, d)
    key_folded = _affinity_key(tokens, W_sel)
    weights, assign = _route(key_folded, c)
    return (weights, assign, cap)
